# baseline (speedup 1.0000x reference)
import functools

import jax
import jax.numpy as jnp
from jax import lax
from jax.experimental import pallas as pl
from jax.experimental.pallas import tpu as pltpu
from jax.experimental.pallas import tpu_sc as plsc

_BATCH = 64
_SEQ = 20
_D = 49152
_N = _BATCH * _SEQ
_NC = 2
_NS = 16
_NW = _NC * _NS
_RPW = _N // _NW

_mesh = plsc.VectorSubcoreMesh(core_axis_name="c", subcore_axis_name="s")


@functools.partial(
    pl.kernel,
    out_type=jax.ShapeDtypeStruct((_N, _D), jnp.float32),
    mesh=_mesh,
    scratch_types=[
        pltpu.VMEM((_RPW * 16,), jnp.int32),
        pltpu.SemaphoreType.DMA,
    ],
)
def _sc_gather(idx_hbm, table_hbm, out_hbm, idx_v, sem):
    wid = lax.axis_index("s") * _NC + lax.axis_index("c")
    base = wid * _RPW
    pltpu.sync_copy(idx_hbm.at[pl.ds(base * 16, _RPW * 16)], idx_v)

    def fire(j, carry):
        i = idx_v[pl.ds(pl.multiple_of(16 * j, 8), 16)][0]
        pltpu.async_copy(
            table_hbm.at[pl.ds(i, 1)], out_hbm.at[pl.ds(base + j, 1)], sem
        )
        return carry

    lax.fori_loop(0, _RPW, fire, 0)

    def drain(j, carry):
        pltpu.make_async_copy(
            table_hbm.at[pl.ds(0, 1)], out_hbm.at[pl.ds(base, 1)], sem
        ).wait()
        return carry

    lax.fori_loop(0, _RPW, drain, 0)


def kernel(prefix, embedding_table):
    idx_flat = prefix.reshape(_N).astype(jnp.int32)
    idx16 = jnp.repeat(idx_flat, 16)
    out = _sc_gather(idx16, embedding_table)
    return out.reshape(_BATCH, _SEQ, _D)


# speedup vs baseline: 12.2006x; 12.2006x over previous
import functools

import jax
import jax.numpy as jnp
from jax import lax
from jax.experimental import pallas as pl
from jax.experimental.pallas import tpu as pltpu
from jax.experimental.pallas import tpu_sc as plsc

_BATCH = 64
_SEQ = 20
_D = 49152
_N = _BATCH * _SEQ
_NC = 2
_NS = 16
_NW = _NC * _NS
_RPW = _N // _NW

_mesh = plsc.VectorSubcoreMesh(core_axis_name="c", subcore_axis_name="s")


@functools.partial(
    pl.kernel,
    out_type=jax.ShapeDtypeStruct((_N, _D), jnp.float32),
    mesh=_mesh,
    scratch_types=[
        pltpu.VMEM((_RPW * 8,), jnp.int32),
        pltpu.VMEM((1, _D), jnp.float32),
        pltpu.VMEM((1, _D), jnp.float32),
        pltpu.SemaphoreType.DMA,
        pltpu.SemaphoreType.DMA,
        pltpu.SemaphoreType.DMA,
        pltpu.SemaphoreType.DMA,
    ],
)
def _sc_gather(idx_hbm, table_hbm, out_hbm, idx_v, buf0, buf1, g0, g1, s0, s1):
    wid = lax.axis_index("s") * _NC + lax.axis_index("c")
    base = wid * _RPW
    pltpu.sync_copy(idx_hbm.at[pl.ds(base * 8, _RPW * 8)], idx_v)

    def _idx(j):
        return idx_v.at[pl.ds(pl.multiple_of(8 * j, 8), 1)]

    def start_gather(j, buf, sem):
        pltpu.async_copy(table_hbm.at[_idx(j)], buf, sem)

    def wait_gather(j, buf, sem):
        pltpu.make_async_copy(table_hbm.at[_idx(j)], buf, sem).wait()

    def start_scatter(j, buf, sem):
        pltpu.async_copy(buf, out_hbm.at[pl.ds(base + j, 1)], sem)

    def wait_scatter(j, buf, sem):
        pltpu.make_async_copy(buf, out_hbm.at[pl.ds(base + j, 1)], sem).wait()

    start_gather(0, buf0, g0)

    def body(t, carry):
        j0 = 2 * t
        j1 = j0 + 1
        wait_gather(j0, buf0, g0)
        start_scatter(j0, buf0, s0)

        @pl.when(t > 0)
        def _():
            wait_scatter(j1 - 2, buf1, s1)

        start_gather(j1, buf1, g1)
        wait_gather(j1, buf1, g1)
        start_scatter(j1, buf1, s1)
        wait_scatter(j0, buf0, s0)

        @pl.when(t < _RPW // 2 - 1)
        def _():
            start_gather(j0 + 2, buf0, g0)

        return carry

    lax.fori_loop(0, _RPW // 2, body, 0)
    wait_scatter(_RPW - 1, buf1, s1)


def kernel(prefix, embedding_table):
    idx_flat = prefix.reshape(_N).astype(jnp.int32)
    idx8 = jnp.repeat(idx_flat, 8)
    out = _sc_gather(idx8, embedding_table)
    return out.reshape(_BATCH, _SEQ, _D)
